# baseline (speedup 1.0000x reference)
import numpy as _np

import jax
import jax.numpy as jnp
from jax import lax
from jax.experimental import pallas as pl
from jax.experimental.pallas import tpu as pltpu
from jax.experimental.pallas import tpu_sc as plsc

N = 50000
VOCAB = 512
HID = 64
H1, C1 = 8, 8
H2, C2 = 4, 16
NG = 64
NC = 20

BLK = 128
TAB_ROWS = 50048
NBLK = TAB_ROWS // BLK
QSIZE = 12800
CROWS = QSIZE + BLK
ACC_ROWS = 4 * QSIZE
E_TOT = 800000 + N
GRP = 128
NSUB = 16
G_PER_SUB = -(-E_TOT // (GRP * NSUB))
E_PAD = G_PER_SUB * GRP * NSUB
TG = E_PAD // GRP


def _k1_body(x_ref, wp_ref, bp_ref, w1_ref, as_ref, ad_ref,
             h0_ref, t_ref, adt_ref):
    h0 = jnp.dot(x_ref[...], wp_ref[...], preferred_element_type=jnp.float32) + bp_ref[...]
    hw = jnp.dot(h0, w1_ref[...], preferred_element_type=jnp.float32)
    asrc = jnp.dot(hw, as_ref[...], preferred_element_type=jnp.float32)
    adst = jnp.dot(hw, ad_ref[...], preferred_element_type=jnp.float32)
    h0_ref[...] = h0
    t_ref[...] = jnp.concatenate([hw, asrc], axis=1)
    adt_ref[...] = adst


def _expand_mat(heads, ch):
    hi = lax.broadcasted_iota(jnp.int32, (heads, heads * ch), 0)
    fi = lax.broadcasted_iota(jnp.int32, (heads, heads * ch), 1)
    return (fi // ch == hi).astype(jnp.float32)


def _gat_post(acc, den, heads, ch, b, hprev, g, be):
    denr = jnp.dot(den[:, :heads], _expand_mat(heads, ch),
                   preferred_element_type=jnp.float32)
    out = acc / (denr + 1e-16) + b
    a = jnp.where(out > 0, out, jnp.exp(out) - 1.0)
    h = a + hprev
    m = h.mean(axis=1, keepdims=True)
    v = ((h - m) ** 2).mean(axis=1, keepdims=True)
    return (h - m) / jnp.sqrt(v + 1e-5) * g + be


def _mid_body(acc_ref, den_ref, hprev_ref, b_ref, g_ref, be_ref,
              w_ref, as_ref, ad_ref, hout_ref, t_ref, adt_ref):
    h = _gat_post(acc_ref[...], den_ref[...], H1, C1, b_ref[...],
                  hprev_ref[...], g_ref[...], be_ref[...])
    hout_ref[...] = h
    hw = jnp.dot(h, w_ref[...], preferred_element_type=jnp.float32)
    asrc = jnp.dot(hw, as_ref[...], preferred_element_type=jnp.float32)
    adst = jnp.dot(hw, ad_ref[...], preferred_element_type=jnp.float32)
    t_ref[...] = jnp.concatenate([hw, asrc], axis=1)
    adt_ref[...] = adst


def _k5_body(acc_ref, den_ref, hprev_ref, h0_ref, b_ref, g_ref, be_ref,
             batch_ref, s2_ref, s0_ref, cnt_ref):
    i = pl.program_id(0)
    h2 = _gat_post(acc_ref[...], den_ref[...], H2, C2, b_ref[...],
                   hprev_ref[...], g_ref[...], be_ref[...])
    bt = batch_ref[0]
    oht = (lax.broadcasted_iota(jnp.int32, (NG, BLK), 0)
           == jnp.broadcast_to(bt, (NG, BLK))).astype(jnp.float32)
    p2 = jnp.dot(oht, h2, preferred_element_type=jnp.float32)
    p0 = jnp.dot(oht, h0_ref[...], preferred_element_type=jnp.float32)
    pc = jnp.dot(oht, jnp.ones((BLK, HID), jnp.float32),
                 preferred_element_type=jnp.float32)

    @pl.when(i == 0)
    def _():
        s2_ref[...] = p2
        s0_ref[...] = p0
        cnt_ref[...] = pc

    @pl.when(i != 0)
    def _():
        s2_ref[...] += p2
        s0_ref[...] += p0
        cnt_ref[...] += pc


def _k6_body(s2_ref, s0_ref, cnt_ref, wf_ref, bf_ref, wc1_ref, bc1_ref,
             wc2_ref, bc2_ref, o_ref):
    cm = jnp.maximum(cnt_ref[...], 1.0)
    z = jnp.concatenate([s2_ref[...] / cm, s0_ref[...] / cm], axis=1)
    z = jnp.maximum(jnp.dot(z, wf_ref[...], preferred_element_type=jnp.float32)
                    + bf_ref[...], 0.0)
    z = jnp.maximum(jnp.dot(z, wc1_ref[...], preferred_element_type=jnp.float32)
                    + bc1_ref[...], 0.0)
    z = jnp.dot(z, wc2_ref[...], preferred_element_type=jnp.float32) + bc2_ref[...]
    mx = jnp.max(z, axis=1, keepdims=True)
    lse = jnp.log(jnp.sum(jnp.exp(z - mx), axis=1, keepdims=True)) + mx
    o_ref[...] = z - lse


_GDN = lax.GatherDimensionNumbers(
    offset_dims=(), collapsed_slice_dims=(0,), start_index_map=(0,))


def _make_edge_kernel(ch):
    csh = ch.bit_length() - 1

    def body(src2d, dst2d, table, adt, acc_out, den_out,
             srcb0, srcb1, dstb0, dstb1, dstl0, dstl1, grow0, grow1,
             adstb0, adstb1, outb0, outb1, wbuf0, wbuf1,
             gsem, isem, ssem, acc_s, den_s):
        srcb_ = (srcb0, srcb1)
        dstb_ = (dstb0, dstb1)
        dstl_ = (dstl0, dstl1)
        grow_ = (grow0, grow1)
        adstb_ = (adstb0, adstb1)
        outb_ = (outb0, outb1)
        wbuf_ = (wbuf0, wbuf1)
        outb = outb0
        wbuf = wbuf0
        c = lax.axis_index("c")
        s = lax.axis_index("s")
        lane = lax.iota(jnp.int32, 16)
        zero16 = jnp.zeros((16,), jnp.float32)

        def _wexpand(w, j):
            idx = ((j * 16 + lane) >> csh).reshape(16, 1)
            return lax.gather(w, idx, _GDN, (1,),
                              mode=lax.GatherScatterMode.PROMISE_IN_BOUNDS)

        for p in range(2):
            q = c + 2 * p
            def _zrow(r, _):
                for j in range(4):
                    outb[r, pl.ds(j * 16, 16)] = zero16
                wbuf[r, pl.ds(0, 16)] = zero16
                return 0
            lax.fori_loop(0, GRP, _zrow, 0)
            zbase = s * (CROWS // NSUB)
            def _zcp(r, _):
                pltpu.sync_copy(outb, acc_s.at[pl.ds(zbase + r * GRP, GRP)])
                pltpu.sync_copy(wbuf, den_s.at[pl.ds(zbase + r * GRP, GRP)])
                return 0
            nfull = (CROWS // NSUB) // GRP
            lax.fori_loop(0, nfull, _zcp, 0)
            rem = CROWS // NSUB - nfull * GRP
            pltpu.sync_copy(outb.at[pl.ds(0, rem)],
                            acc_s.at[pl.ds(zbase + nfull * GRP, rem)])
            pltpu.sync_copy(wbuf.at[pl.ds(0, rem)],
                            den_s.at[pl.ds(zbase + nfull * GRP, rem)])
            plsc.subcore_barrier()

            G = G_PER_SUB
            base_row = s * G

            def _start_gather(b):
                pltpu.async_copy(table.at[srcb_[b]], grow_[b], gsem)
                pltpu.async_copy(adt.at[dstb_[b]], adstb_[b], gsem)

            def _start_idx(b, g_next):
                rw = base_row + g_next
                pltpu.async_copy(src2d.at[rw], srcb_[b], isem)
                pltpu.async_copy(dst2d.at[rw], dstb_[b], isem)

            def _wait_idx(b):
                pltpu.make_async_copy(src2d.at[base_row], srcb_[b], isem).wait()
                pltpu.make_async_copy(dst2d.at[base_row], dstb_[b], isem).wait()

            def _wait_scat(b):
                pltpu.make_async_copy(outb_[b], acc_s.at[dstl_[b]], ssem).wait()
                pltpu.make_async_copy(wbuf_[b], den_s.at[dstl_[b]], ssem).wait()

            pltpu.sync_copy(src2d.at[base_row], srcb_[0])
            pltpu.sync_copy(dst2d.at[base_row], dstb_[0])
            _start_gather(0)
            _start_idx(1, 1)

            def _leg(g, b):
                grow, adstb = grow_[b], adstb_[b]
                outb, wbuf, dstl, dstb = outb_[b], wbuf_[b], dstl_[b], dstb_[b]
                pltpu.make_async_copy(table.at[srcb_[b]], grow, gsem).wait()
                pltpu.make_async_copy(adt.at[dstb], adstb, gsem).wait()

                @pl.when(g >= 2)
                def _():
                    _wait_scat(b)

                def _loc(k, _):
                    d16 = dstb[pl.ds(k * 16, 16)]
                    ld = d16 - q * QSIZE
                    oob = (ld < 0) | (ld >= QSIZE)
                    dstl[pl.ds(k * 16, 16)] = jnp.where(oob, QSIZE, ld)
                    return 0
                lax.fori_loop(0, GRP // 16, _loc, 0)

                @pl.when(g + 1 < G)
                def _():
                    _wait_idx(1 - b)
                    _start_gather(1 - b)

                @pl.when(g + 2 < G)
                def _():
                    _start_idx(b, g + 2)

                def _edge(e2, _):
                    for u in range(2):
                        e = 2 * e2 + u
                        av = grow[e, pl.ds(HID, 16)]
                        dv = adstb[e, pl.ds(0, 16)]
                        sv = av + dv
                        w = jnp.exp(jnp.maximum(sv, sv * 0.2))
                        wbuf[e, pl.ds(0, 16)] = w
                        for j in range(4):
                            hv = grow[e, pl.ds(j * 16, 16)]
                            outb[e, pl.ds(j * 16, 16)] = hv * _wexpand(w, j)
                    return 0
                lax.fori_loop(0, GRP // 2, _edge, 0)
                pltpu.async_copy(outb, acc_s.at[dstl], ssem, add=True)
                pltpu.async_copy(wbuf, den_s.at[dstl], ssem, add=True)

            def _pair(t, _):
                _leg(2 * t, 0)
                _leg(2 * t + 1, 1)
                return 0
            lax.fori_loop(0, G // 2, _pair, 0)
            _wait_scat(0)
            _wait_scat(1)
            plsc.subcore_barrier()
            nr = QSIZE // NSUB
            pltpu.sync_copy(acc_s.at[pl.ds(s * nr, nr)],
                            acc_out.at[pl.ds(q * QSIZE + s * nr, nr)])
            pltpu.sync_copy(den_s.at[pl.ds(s * nr, nr)],
                            den_out.at[pl.ds(q * QSIZE + s * nr, nr)])
            plsc.subcore_barrier()

    mesh = plsc.VectorSubcoreMesh(core_axis_name="c", subcore_axis_name="s")
    return pl.kernel(
        body,
        compiler_params=pltpu.CompilerParams(use_tc_tiling_on_sc=False),
        out_type=[jax.ShapeDtypeStruct((ACC_ROWS, HID), jnp.float32),
                  jax.ShapeDtypeStruct((ACC_ROWS, 16), jnp.float32)],
        mesh=mesh,
        scratch_types=(
            [pltpu.VMEM((GRP,), jnp.int32)] * 6
            + [pltpu.VMEM((GRP, HID + 16), jnp.float32)] * 2
            + [pltpu.VMEM((GRP, 16), jnp.float32)] * 2
            + [pltpu.VMEM((GRP, HID), jnp.float32)] * 2
            + [pltpu.VMEM((GRP, 16), jnp.float32)] * 2
            + [pltpu.SemaphoreType.DMA] * 3
            + [pltpu.VMEM_SHARED((CROWS, HID), jnp.float32),
               pltpu.VMEM_SHARED((CROWS, 16), jnp.float32)]
        ),
    )


_edge_l1 = _make_edge_kernel(C1)
_edge_l2 = _make_edge_kernel(C2)


def _blockdiag(a, heads, ch):
    flat = a.reshape(heads * ch)
    out = jnp.zeros((HID, 16), jnp.float32)
    return out.at[jnp.arange(HID), jnp.arange(HID) // ch].set(flat)


def kernel(x, edge_index, batch, Wp, bp, W1, as1, ad1, b1, g1, be1,
           W2, as2, ad2, b2, g2, be2, Wf, bf, Wc1, bc1, Wc2, bc2):
    f32 = jnp.float32
    x = x.astype(f32)
    loops = jnp.arange(N, dtype=jnp.int32)
    src = jnp.concatenate([edge_index[0].astype(jnp.int32), loops])
    dst = jnp.concatenate([edge_index[1].astype(jnp.int32), loops])
    pad = E_PAD - E_TOT
    src = jnp.concatenate([src, jnp.full((pad,), N, jnp.int32)]).reshape(TG, GRP)
    dst = jnp.concatenate([dst, jnp.full((pad,), N, jnp.int32)]).reshape(TG, GRP)

    xp = jnp.pad(x, ((0, TAB_ROWS - N), (0, 0)))
    batchp = jnp.pad(batch.astype(jnp.int32), (0, TAB_ROWS - N),
                     constant_values=NG).reshape(NBLK, 1, BLK)

    as1p, ad1p = _blockdiag(as1, H1, C1), _blockdiag(ad1, H1, C1)
    as2p, ad2p = _blockdiag(as2, H2, C2), _blockdiag(ad2, H2, C2)
    row = lambda v: v.reshape(1, -1).astype(f32)

    full = lambda shp: pl.BlockSpec(shp, lambda i: tuple(0 for _ in shp))
    rowblk = lambda w: pl.BlockSpec((BLK, w), lambda i: (i, 0))

    h0, t1, adt1 = pl.pallas_call(
        _k1_body,
        grid=(NBLK,),
        in_specs=[rowblk(VOCAB), full((VOCAB, HID)), full((1, HID)),
                  full((HID, HID)), full((HID, 16)), full((HID, 16))],
        out_specs=[rowblk(HID), rowblk(HID + 16), rowblk(16)],
        out_shape=[jax.ShapeDtypeStruct((TAB_ROWS, HID), f32),
                   jax.ShapeDtypeStruct((TAB_ROWS, HID + 16), f32),
                   jax.ShapeDtypeStruct((TAB_ROWS, 16), f32)],
    )(xp, Wp.astype(f32), row(bp), W1.astype(f32), as1p, ad1p)

    acc1, den1 = _edge_l1(src, dst, t1, adt1)

    h1, t2, adt2 = pl.pallas_call(
        _mid_body,
        grid=(NBLK,),
        in_specs=[rowblk(HID), rowblk(16), rowblk(HID), full((1, HID)),
                  full((1, HID)), full((1, HID)), full((HID, HID)),
                  full((HID, 16)), full((HID, 16))],
        out_specs=[rowblk(HID), rowblk(HID + 16), rowblk(16)],
        out_shape=[jax.ShapeDtypeStruct((TAB_ROWS, HID), f32),
                   jax.ShapeDtypeStruct((TAB_ROWS, HID + 16), f32),
                   jax.ShapeDtypeStruct((TAB_ROWS, 16), f32)],
    )(acc1, den1, h0, row(b1), row(g1), row(be1), W2.astype(f32), as2p, ad2p)

    acc2, den2 = _edge_l2(src, dst, t2, adt2)

    s2, s0, cnt = pl.pallas_call(
        _k5_body,
        grid=(NBLK,),
        in_specs=[rowblk(HID), rowblk(16), rowblk(HID), rowblk(HID),
                  full((1, HID)), full((1, HID)), full((1, HID)),
                  pl.BlockSpec((1, 1, BLK), lambda i: (i, 0, 0))],
        out_specs=[full((NG, HID)), full((NG, HID)), full((NG, HID))],
        out_shape=[jax.ShapeDtypeStruct((NG, HID), f32)] * 3,
    )(acc2, den2, h1, h0, row(b2), row(g2), row(be2), batchp)

    return pl.pallas_call(
        _k6_body,
        out_shape=jax.ShapeDtypeStruct((NG, NC), f32),
    )(s2, s0, cnt, Wf.astype(f32), row(bf), Wc1.astype(f32), row(bc1),
      Wc2.astype(f32), row(bc2))


# speedup vs baseline: 1.5679x; 1.5679x over previous
import numpy as _np

import jax
import jax.numpy as jnp
from jax import lax
from jax.experimental import pallas as pl
from jax.experimental.pallas import tpu as pltpu
from jax.experimental.pallas import tpu_sc as plsc

N = 50000
VOCAB = 512
HID = 64
H1, C1 = 8, 8
H2, C2 = 4, 16
NG = 64
NC = 20

BLK = 128
TAB_ROWS = 50048
NBLK = TAB_ROWS // BLK
QSIZE = 12800
CROWS = QSIZE + BLK
ACC_ROWS = 4 * QSIZE
E_TOT = 800000 + N
GRP = 128
NSUB = 16
G_PER_SUB = -(-E_TOT // (GRP * NSUB))
E_PAD = G_PER_SUB * GRP * NSUB
TG = E_PAD // GRP


def _k1_body(x_ref, wp_ref, bp_ref, w1_ref, as_ref, ad_ref,
             h0_ref, t_ref, adt_ref):
    h0 = jnp.dot(x_ref[...], wp_ref[...], preferred_element_type=jnp.float32) + bp_ref[...]
    hw = jnp.dot(h0, w1_ref[...], preferred_element_type=jnp.float32)
    asrc = jnp.dot(hw, as_ref[...], preferred_element_type=jnp.float32)
    adst = jnp.dot(hw, ad_ref[...], preferred_element_type=jnp.float32)
    h0_ref[...] = h0
    t_ref[...] = jnp.concatenate([hw, asrc], axis=1)
    adt_ref[...] = adst


def _expand_mat(heads, ch):
    hi = lax.broadcasted_iota(jnp.int32, (heads, heads * ch), 0)
    fi = lax.broadcasted_iota(jnp.int32, (heads, heads * ch), 1)
    return (fi // ch == hi).astype(jnp.float32)


def _gat_post(accden, heads, ch, b, hprev, g, be):
    denr = jnp.dot(accden[:, HID:HID + heads], _expand_mat(heads, ch),
                   preferred_element_type=jnp.float32)
    out = accden[:, :HID] / (denr + 1e-16) + b
    a = jnp.where(out > 0, out, jnp.exp(out) - 1.0)
    h = a + hprev
    m = h.mean(axis=1, keepdims=True)
    v = ((h - m) ** 2).mean(axis=1, keepdims=True)
    return (h - m) / jnp.sqrt(v + 1e-5) * g + be


def _mid_body(acc_ref, hprev_ref, b_ref, g_ref, be_ref,
              w_ref, as_ref, ad_ref, hout_ref, t_ref, adt_ref):
    h = _gat_post(acc_ref[...], H1, C1, b_ref[...],
                  hprev_ref[...], g_ref[...], be_ref[...])
    hout_ref[...] = h
    hw = jnp.dot(h, w_ref[...], preferred_element_type=jnp.float32)
    asrc = jnp.dot(hw, as_ref[...], preferred_element_type=jnp.float32)
    adst = jnp.dot(hw, ad_ref[...], preferred_element_type=jnp.float32)
    t_ref[...] = jnp.concatenate([hw, asrc], axis=1)
    adt_ref[...] = adst


def _k5_body(acc_ref, hprev_ref, h0_ref, b_ref, g_ref, be_ref,
             batch_ref, s2_ref, s0_ref, cnt_ref):
    i = pl.program_id(0)
    h2 = _gat_post(acc_ref[...], H2, C2, b_ref[...],
                   hprev_ref[...], g_ref[...], be_ref[...])
    bt = batch_ref[0]
    oht = (lax.broadcasted_iota(jnp.int32, (NG, BLK), 0)
           == jnp.broadcast_to(bt, (NG, BLK))).astype(jnp.float32)
    p2 = jnp.dot(oht, h2, preferred_element_type=jnp.float32)
    p0 = jnp.dot(oht, h0_ref[...], preferred_element_type=jnp.float32)
    pc = jnp.dot(oht, jnp.ones((BLK, HID), jnp.float32),
                 preferred_element_type=jnp.float32)

    @pl.when(i == 0)
    def _():
        s2_ref[...] = p2
        s0_ref[...] = p0
        cnt_ref[...] = pc

    @pl.when(i != 0)
    def _():
        s2_ref[...] += p2
        s0_ref[...] += p0
        cnt_ref[...] += pc


def _k6_body(s2_ref, s0_ref, cnt_ref, wf_ref, bf_ref, wc1_ref, bc1_ref,
             wc2_ref, bc2_ref, o_ref):
    cm = jnp.maximum(cnt_ref[...], 1.0)
    z = jnp.concatenate([s2_ref[...] / cm, s0_ref[...] / cm], axis=1)
    z = jnp.maximum(jnp.dot(z, wf_ref[...], preferred_element_type=jnp.float32)
                    + bf_ref[...], 0.0)
    z = jnp.maximum(jnp.dot(z, wc1_ref[...], preferred_element_type=jnp.float32)
                    + bc1_ref[...], 0.0)
    z = jnp.dot(z, wc2_ref[...], preferred_element_type=jnp.float32) + bc2_ref[...]
    mx = jnp.max(z, axis=1, keepdims=True)
    lse = jnp.log(jnp.sum(jnp.exp(z - mx), axis=1, keepdims=True)) + mx
    o_ref[...] = z - lse


_GDN = lax.GatherDimensionNumbers(
    offset_dims=(), collapsed_slice_dims=(0,), start_index_map=(0,))


def _make_edge_kernel(ch):
    csh = ch.bit_length() - 1

    def body(sd3, table, adt, acc_out,
             sdb0, sdb1, dstl0, dstl1, grow0, grow1,
             adstb0, adstb1, outb0, outb1,
             gsem, isem, ssem, acc_s):
        sdb_ = (sdb0, sdb1)
        dstl_ = (dstl0, dstl1)
        grow_ = (grow0, grow1)
        adstb_ = (adstb0, adstb1)
        outb_ = (outb0, outb1)
        outb = outb0
        c = lax.axis_index("c")
        s = lax.axis_index("s")
        lane = lax.iota(jnp.int32, 16)
        zero16 = jnp.zeros((16,), jnp.float32)

        def _wexpand(w, j):
            idx = ((j * 16 + lane) >> csh).reshape(16, 1)
            return lax.gather(w, idx, _GDN, (1,),
                              mode=lax.GatherScatterMode.PROMISE_IN_BOUNDS)

        for p in range(2):
            q = c + 2 * p
            def _zrow(r, _):
                for j in range(5):
                    outb[r, pl.ds(j * 16, 16)] = zero16
                return 0
            lax.fori_loop(0, GRP, _zrow, 0)
            zbase = s * (CROWS // NSUB)
            def _zcp(r, _):
                pltpu.sync_copy(outb, acc_s.at[pl.ds(zbase + r * GRP, GRP)])
                return 0
            nfull = (CROWS // NSUB) // GRP
            lax.fori_loop(0, nfull, _zcp, 0)
            rem = CROWS // NSUB - nfull * GRP
            pltpu.sync_copy(outb.at[pl.ds(0, rem)],
                            acc_s.at[pl.ds(zbase + nfull * GRP, rem)])
            plsc.subcore_barrier()

            G = G_PER_SUB
            base_row = s * G

            def _start_gather(b):
                pltpu.async_copy(table.at[sdb_[b].at[0]], grow_[b], gsem)
                pltpu.async_copy(adt.at[sdb_[b].at[1]], adstb_[b], gsem)

            def _start_idx(b, g_next):
                pltpu.async_copy(sd3.at[base_row + g_next], sdb_[b], isem)

            def _wait_idx(b):
                pltpu.make_async_copy(sd3.at[base_row], sdb_[b], isem).wait()

            def _wait_scat(b):
                pltpu.make_async_copy(outb_[b], acc_s.at[dstl_[b]], ssem).wait()

            pltpu.sync_copy(sd3.at[base_row], sdb_[0])
            _start_gather(0)
            _start_idx(1, 1)

            def _leg(g, b):
                grow, adstb = grow_[b], adstb_[b]
                outb, dstl, sdb = outb_[b], dstl_[b], sdb_[b]
                pltpu.make_async_copy(table.at[sdb.at[0]], grow, gsem).wait()
                pltpu.make_async_copy(adt.at[sdb.at[1]], adstb, gsem).wait()

                @pl.when(g >= 2)
                def _():
                    _wait_scat(b)

                def _loc(k, _):
                    d16 = sdb[1, pl.ds(k * 16, 16)]
                    ld = d16 - q * QSIZE
                    oob = (ld < 0) | (ld >= QSIZE)
                    dstl[pl.ds(k * 16, 16)] = jnp.where(oob, QSIZE, ld)
                    return 0
                lax.fori_loop(0, GRP // 16, _loc, 0)

                @pl.when(g + 1 < G)
                def _():
                    _wait_idx(1 - b)
                    _start_gather(1 - b)

                @pl.when(g + 2 < G)
                def _():
                    _start_idx(b, g + 2)

                def _edge(e2, _):
                    for u in range(2):
                        e = 2 * e2 + u
                        av = grow[e, pl.ds(HID, 16)]
                        dv = adstb[e, pl.ds(0, 16)]
                        sv = av + dv
                        w = jnp.exp(jnp.maximum(sv, sv * 0.2))
                        outb[e, pl.ds(HID, 16)] = w
                        for j in range(4):
                            hv = grow[e, pl.ds(j * 16, 16)]
                            outb[e, pl.ds(j * 16, 16)] = hv * _wexpand(w, j)
                    return 0
                lax.fori_loop(0, GRP // 2, _edge, 0)
                pltpu.async_copy(outb, acc_s.at[dstl], ssem, add=True)

            def _pair(t, _):
                _leg(2 * t, 0)
                _leg(2 * t + 1, 1)
                return 0
            lax.fori_loop(0, G // 2, _pair, 0)
            _wait_scat(0)
            _wait_scat(1)
            plsc.subcore_barrier()
            nr = QSIZE // NSUB
            pltpu.sync_copy(acc_s.at[pl.ds(s * nr, nr)],
                            acc_out.at[pl.ds(q * QSIZE + s * nr, nr)])
            plsc.subcore_barrier()

    mesh = plsc.VectorSubcoreMesh(core_axis_name="c", subcore_axis_name="s")
    return pl.kernel(
        body,
        compiler_params=pltpu.CompilerParams(use_tc_tiling_on_sc=False),
        out_type=jax.ShapeDtypeStruct((ACC_ROWS, HID + 16), jnp.float32),
        mesh=mesh,
        scratch_types=(
            [pltpu.VMEM((2, GRP), jnp.int32)] * 2
            + [pltpu.VMEM((GRP,), jnp.int32)] * 2
            + [pltpu.VMEM((GRP, HID + 16), jnp.float32)] * 2
            + [pltpu.VMEM((GRP, 16), jnp.float32)] * 2
            + [pltpu.VMEM((GRP, HID + 16), jnp.float32)] * 2
            + [pltpu.SemaphoreType.DMA] * 3
            + [pltpu.VMEM_SHARED((CROWS, HID + 16), jnp.float32)]
        ),
    )


_edge_l1 = _make_edge_kernel(C1)
_edge_l2 = _make_edge_kernel(C2)


def _blockdiag(a, heads, ch):
    flat = a.reshape(heads * ch)
    out = jnp.zeros((HID, 16), jnp.float32)
    return out.at[jnp.arange(HID), jnp.arange(HID) // ch].set(flat)


def kernel(x, edge_index, batch, Wp, bp, W1, as1, ad1, b1, g1, be1,
           W2, as2, ad2, b2, g2, be2, Wf, bf, Wc1, bc1, Wc2, bc2):
    f32 = jnp.float32
    x = x.astype(f32)
    loops = jnp.arange(N, dtype=jnp.int32)
    src = jnp.concatenate([edge_index[0].astype(jnp.int32), loops])
    dst = jnp.concatenate([edge_index[1].astype(jnp.int32), loops])
    pad = E_PAD - E_TOT
    src = jnp.concatenate([src, jnp.full((pad,), N, jnp.int32)]).reshape(TG, GRP)
    dst = jnp.concatenate([dst, jnp.full((pad,), N, jnp.int32)]).reshape(TG, GRP)
    sd3 = jnp.stack([src, dst], axis=1)

    xp = jnp.pad(x, ((0, TAB_ROWS - N), (0, 0)))
    batchp = jnp.pad(batch.astype(jnp.int32), (0, TAB_ROWS - N),
                     constant_values=NG).reshape(NBLK, 1, BLK)

    as1p, ad1p = _blockdiag(as1, H1, C1), _blockdiag(ad1, H1, C1)
    as2p, ad2p = _blockdiag(as2, H2, C2), _blockdiag(ad2, H2, C2)
    row = lambda v: v.reshape(1, -1).astype(f32)

    full = lambda shp: pl.BlockSpec(shp, lambda i: tuple(0 for _ in shp))
    rowblk = lambda w: pl.BlockSpec((BLK, w), lambda i: (i, 0))

    h0, t1, adt1 = pl.pallas_call(
        _k1_body,
        grid=(NBLK,),
        in_specs=[rowblk(VOCAB), full((VOCAB, HID)), full((1, HID)),
                  full((HID, HID)), full((HID, 16)), full((HID, 16))],
        out_specs=[rowblk(HID), rowblk(HID + 16), rowblk(16)],
        out_shape=[jax.ShapeDtypeStruct((TAB_ROWS, HID), f32),
                   jax.ShapeDtypeStruct((TAB_ROWS, HID + 16), f32),
                   jax.ShapeDtypeStruct((TAB_ROWS, 16), f32)],
    )(xp, Wp.astype(f32), row(bp), W1.astype(f32), as1p, ad1p)

    acc1 = _edge_l1(sd3, t1, adt1)

    h1, t2, adt2 = pl.pallas_call(
        _mid_body,
        grid=(NBLK,),
        in_specs=[rowblk(HID + 16), rowblk(HID), full((1, HID)),
                  full((1, HID)), full((1, HID)), full((HID, HID)),
                  full((HID, 16)), full((HID, 16))],
        out_specs=[rowblk(HID), rowblk(HID + 16), rowblk(16)],
        out_shape=[jax.ShapeDtypeStruct((TAB_ROWS, HID), f32),
                   jax.ShapeDtypeStruct((TAB_ROWS, HID + 16), f32),
                   jax.ShapeDtypeStruct((TAB_ROWS, 16), f32)],
    )(acc1, h0, row(b1), row(g1), row(be1), W2.astype(f32), as2p, ad2p)

    acc2 = _edge_l2(sd3, t2, adt2)

    s2, s0, cnt = pl.pallas_call(
        _k5_body,
        grid=(NBLK,),
        in_specs=[rowblk(HID + 16), rowblk(HID), rowblk(HID),
                  full((1, HID)), full((1, HID)), full((1, HID)),
                  pl.BlockSpec((1, 1, BLK), lambda i: (i, 0, 0))],
        out_specs=[full((NG, HID)), full((NG, HID)), full((NG, HID))],
        out_shape=[jax.ShapeDtypeStruct((NG, HID), f32)] * 3,
    )(acc2, h1, h0, row(b2), row(g2), row(be2), batchp)

    return pl.pallas_call(
        _k6_body,
        out_shape=jax.ShapeDtypeStruct((NG, NC), f32),
    )(s2, s0, cnt, Wf.astype(f32), row(bf), Wc1.astype(f32), row(bc1),
      Wc2.astype(f32), row(bc2))
